# baseline (speedup 1.0000x reference)
import functools

import jax
import jax.numpy as jnp
from jax import lax
from jax.experimental import pallas as pl
from jax.experimental.pallas import tpu as pltpu
from jax.experimental.pallas import tpu_sc as plsc

NC = 2
NS = 16
CH = 128
NW = NC * NS
CF = 0
FRAC = 0.77


def _rows_acc(N):
  return ((N + 1 + NS * 8 - 1) // (NS * 8)) * 8


def _make_deg(N, nch, nch_s, cf):
  rows_acc = _rows_acc(N)
  nacc = rows_acc * NS
  mesh = plsc.VectorSubcoreMesh(core_axis_name="c", subcore_axis_name="s",
                                num_cores=NC, num_subcores=NS)

  @functools.partial(
      pl.kernel, mesh=mesh,
      compiler_params=pltpu.CompilerParams(use_tc_tiling_on_sc=False),
      out_type=jax.ShapeDtypeStruct((NC, 2, nacc, 8), jnp.float32),
      scratch_types=[
          pltpu.VMEM((nch, CH), jnp.int32),
          pltpu.VMEM((nch, CH), jnp.int32),
          pltpu.VMEM((CH, 8), jnp.float32),
          pltpu.VMEM((rows_acc, 8), jnp.float32),
          pltpu.VMEM_SHARED((nacc, 8), jnp.float32),
          pltpu.VMEM_SHARED((nacc, 8), jnp.float32),
      ],
  )
  def deg(sd_hbm, dd_hbm, ones_hbm, zer_hbm, out_hbm,
          sidx, didx, obuf, zbuf, acc_s, acc_d):
    c = lax.axis_index("c")
    s = lax.axis_index("s")
    wid = s * NC + c
    my_nch = jnp.where(c == cf, nch, nch_s)
    pltpu.sync_copy(ones_hbm, obuf)
    pltpu.sync_copy(zer_hbm, zbuf)
    base = s * rows_acc
    pltpu.sync_copy(zbuf, acc_s.at[pl.ds(base, rows_acc)])
    pltpu.sync_copy(zbuf, acc_d.at[pl.ds(base, rows_acc)])
    plsc.subcore_barrier()
    pltpu.sync_copy(sd_hbm.at[wid], sidx)
    pltpu.sync_copy(dd_hbm.at[wid], didx)

    def chunk(j, _):
      pltpu.sync_copy(obuf, acc_s.at[sidx.at[j]], add=True)
      pltpu.sync_copy(obuf, acc_d.at[didx.at[j]], add=True)
      return 0

    lax.fori_loop(0, my_nch, chunk, 0)
    plsc.subcore_barrier()
    pltpu.sync_copy(acc_s.at[pl.ds(base, rows_acc)], zbuf)
    pltpu.sync_copy(zbuf, out_hbm.at[c, 0, pl.ds(base, rows_acc)])
    pltpu.sync_copy(acc_d.at[pl.ds(base, rows_acc)], zbuf)
    pltpu.sync_copy(zbuf, out_hbm.at[c, 1, pl.ds(base, rows_acc)])

  return deg


def _make_agg(N, nch, nch_s, cf, D):
  rows_acc = _rows_acc(N)
  nacc = rows_acc * NS
  mesh = plsc.VectorSubcoreMesh(core_axis_name="c", subcore_axis_name="s",
                                num_cores=NC, num_subcores=NS)

  @functools.partial(
      pl.kernel, mesh=mesh,
      compiler_params=pltpu.CompilerParams(use_tc_tiling_on_sc=False),
      out_type=jax.ShapeDtypeStruct((NC, nacc, D), jnp.float32),
      scratch_types=[
          pltpu.VMEM((nch, CH), jnp.int32),
          pltpu.VMEM((nch, CH), jnp.int32),
          pltpu.VMEM((CH, D), jnp.float32),
          pltpu.VMEM((CH, D), jnp.float32),
          pltpu.VMEM_SHARED((nacc, D), jnp.float32),
          pltpu.SemaphoreType.DMA,
          pltpu.SemaphoreType.DMA,
      ],
  )
  def agg(y_hbm, sg_hbm, dd_hbm, out_hbm,
          sidx, didx, bufa, bufb, acc, sema, semb):
    c = lax.axis_index("c")
    s = lax.axis_index("s")
    wid = s * NC + c
    my_nch = jnp.where(c == cf, nch, nch_s)
    z16 = jnp.zeros((16,), jnp.float32)

    def zrow(r, _):
      for kk in range(D // 16):
        bufa[r, pl.ds(kk * 16, 16)] = z16
      return 0

    lax.fori_loop(0, CH, zrow, 0)
    base = s * rows_acc
    nfull, rem = divmod(rows_acc, CH)
    for k in range(nfull):
      pltpu.sync_copy(bufa, acc.at[pl.ds(base + k * CH, CH)])
    if rem:
      pltpu.sync_copy(bufa.at[pl.ds(0, rem)],
                      acc.at[pl.ds(base + nfull * CH, rem)])
    plsc.subcore_barrier()
    pltpu.sync_copy(sg_hbm.at[wid], sidx)
    pltpu.sync_copy(dd_hbm.at[wid], didx)
    pltpu.async_copy(y_hbm.at[sidx.at[0]], bufa, sema)
    pltpu.async_copy(y_hbm.at[sidx.at[1]], bufb, semb)

    def pair(p, _):
      j = 2 * p
      pltpu.make_async_copy(y_hbm.at[sidx.at[j]], bufa, sema).wait()
      pltpu.sync_copy(bufa, acc.at[didx.at[j]], add=True)
      pltpu.async_copy(y_hbm.at[sidx.at[j + 2]], bufa, sema)
      pltpu.make_async_copy(y_hbm.at[sidx.at[j + 1]], bufb, semb).wait()
      pltpu.sync_copy(bufb, acc.at[didx.at[j + 1]], add=True)
      pltpu.async_copy(y_hbm.at[sidx.at[j + 3]], bufb, semb)
      return 0

    lax.fori_loop(0, my_nch // 2 - 1, pair, 0)
    j = my_nch - 2
    pltpu.make_async_copy(y_hbm.at[sidx.at[j]], bufa, sema).wait()
    pltpu.sync_copy(bufa, acc.at[didx.at[j]], add=True)
    pltpu.make_async_copy(y_hbm.at[sidx.at[j + 1]], bufb, semb).wait()
    pltpu.sync_copy(bufb, acc.at[didx.at[j + 1]], add=True)
    plsc.subcore_barrier()
    for k in range(nfull):
      pltpu.sync_copy(acc.at[pl.ds(base + k * CH, CH)], bufa)
      pltpu.sync_copy(bufa, out_hbm.at[c, pl.ds(base + k * CH, CH)])
    if rem:
      pltpu.sync_copy(acc.at[pl.ds(base + nfull * CH, rem)],
                      bufa.at[pl.ds(0, rem)])
      pltpu.sync_copy(bufa.at[pl.ds(0, rem)],
                      out_hbm.at[c, pl.ds(base + nfull * CH, rem)])

  return agg


def _tc_first(h, WA, WB, degs, br):
  N, HD = h.shape[0], WA.shape[1]

  def body(h_ref, wa_ref, wb_ref, d_ref, ya_ref, yb_ref, ro_ref, ri_ref):
    d = d_ref[...]
    dout = (d[0, 0] + d[1, 0])[:, 0:1]
    din = (d[0, 1] + d[1, 1])[:, 0:1]
    ro = lax.rsqrt(jnp.maximum(dout, 1.0))
    ri = lax.rsqrt(jnp.maximum(din, 1.0))
    x = h_ref[...]
    ya_ref[...] = jnp.dot(x, wa_ref[...], preferred_element_type=jnp.float32) * ro
    yb_ref[...] = jnp.dot(x, wb_ref[...], preferred_element_type=jnp.float32) * ro
    ro_ref[...] = ro
    ri_ref[...] = ri

  Din = h.shape[1]
  return pl.pallas_call(
      body,
      grid=(N // br,),
      in_specs=[
          pl.BlockSpec((br, Din), lambda i: (i, 0)),
          pl.BlockSpec((Din, HD), lambda i: (0, 0)),
          pl.BlockSpec((Din, HD), lambda i: (0, 0)),
          pl.BlockSpec((NC, 2, br, 8), lambda i: (0, 0, i, 0)),
      ],
      out_specs=[
          pl.BlockSpec((br, HD), lambda i: (i, 0)),
          pl.BlockSpec((br, HD), lambda i: (i, 0)),
          pl.BlockSpec((br, 1), lambda i: (i, 0)),
          pl.BlockSpec((br, 1), lambda i: (i, 0)),
      ],
      out_shape=[
          jax.ShapeDtypeStruct((N, HD), jnp.float32),
          jax.ShapeDtypeStruct((N, HD), jnp.float32),
          jax.ShapeDtypeStruct((N, 1), jnp.float32),
          jax.ShapeDtypeStruct((N, 1), jnp.float32),
      ],
  )(h, WA, WB, degs)


def _tc_mid(PA, PB, ri, ro, bA, bB, gA, gB, beA, beB, Wq, br, two_out):
  N, HD = ri.shape[0], PA.shape[2]
  Dtot = 2.0 * HD

  def body(pa_ref, pb_ref, ri_ref, ro_ref, ba_ref, bb_ref, ga_ref, gb_ref,
           bea_ref, beb_ref, *rest):
    w_refs = rest[:len(Wq)]
    y_refs = rest[len(Wq):]
    ri_v = ri_ref[...]
    xA = (pa_ref[0] + pa_ref[1]) * ri_v + ba_ref[...]
    xB = (pb_ref[0] + pb_ref[1]) * ri_v + bb_ref[...]
    mu = (jnp.sum(xA, -1, keepdims=True) + jnp.sum(xB, -1, keepdims=True)) / Dtot
    dA = xA - mu
    dB = xB - mu
    var = (jnp.sum(dA * dA, -1, keepdims=True)
           + jnp.sum(dB * dB, -1, keepdims=True)) / Dtot
    rs = lax.rsqrt(var + 1e-5)
    xA = jnp.maximum(dA * rs * ga_ref[...] + bea_ref[...], 0.0)
    xB = jnp.maximum(dB * rs * gb_ref[...] + beb_ref[...], 0.0)
    ro_v = ro_ref[...]
    y_refs[0][...] = (jnp.dot(xA, w_refs[0][...], preferred_element_type=jnp.float32)
                      + jnp.dot(xB, w_refs[1][...], preferred_element_type=jnp.float32)) * ro_v
    if two_out:
      y_refs[1][...] = (jnp.dot(xA, w_refs[2][...], preferred_element_type=jnp.float32)
                        + jnp.dot(xB, w_refs[3][...], preferred_element_type=jnp.float32)) * ro_v

  n_out = 2 if two_out else 1
  Dn = Wq[0].shape[1]
  return pl.pallas_call(
      body,
      grid=(N // br,),
      in_specs=(
          [pl.BlockSpec((NC, br, HD), lambda i: (0, i, 0))] * 2
          + [pl.BlockSpec((br, 1), lambda i: (i, 0))] * 2
          + [pl.BlockSpec((1, HD), lambda i: (0, 0))] * 6
          + [pl.BlockSpec((HD, Dn), lambda i: (0, 0))] * len(Wq)
      ),
      out_specs=[pl.BlockSpec((br, Dn), lambda i: (i, 0))] * n_out,
      out_shape=[jax.ShapeDtypeStruct((N, Dn), jnp.float32)] * n_out,
  )(PA, PB, ri, ro, bA.reshape(1, HD), bB.reshape(1, HD),
    gA.reshape(1, HD), gB.reshape(1, HD), beA.reshape(1, HD),
    beB.reshape(1, HD), *Wq)


def _tc_last(P, ri, b2, br):
  N, D = ri.shape[0], P.shape[2]

  def body(p_ref, ri_ref, b_ref, y_ref):
    y_ref[...] = (p_ref[0] + p_ref[1]) * ri_ref[...] + b_ref[...]

  return pl.pallas_call(
      body,
      grid=(N // br,),
      in_specs=[
          pl.BlockSpec((NC, br, D), lambda i: (0, i, 0)),
          pl.BlockSpec((br, 1), lambda i: (i, 0)),
          pl.BlockSpec((1, D), lambda i: (0, 0)),
      ],
      out_specs=pl.BlockSpec((br, D), lambda i: (i, 0)),
      out_shape=jax.ShapeDtypeStruct((N, D), jnp.float32),
  )(P, ri, b2.reshape(1, D))


def kernel(h, edge_index, W0, b0, g0, be0, W1, b1, g1, be1, W2, b2):
  N = h.shape[0]
  E = edge_index.shape[1]
  nch_tot = -(-E // (NS * CH)) + 1
  nch_tot += nch_tot % 2
  nch_f = int(nch_tot * FRAC) // 2 * 2
  nch_s = nch_tot - nch_f
  cap_f, cap_s = nch_f * CH, nch_s * CH
  F = NS * cap_f
  src = edge_index[0]
  dst = edge_index[1]

  def _split(a, padval):
    fast = a[:F].reshape(NS, cap_f)
    slow = jnp.concatenate(
        [a[F:], jnp.full((NS * cap_s - (E - F),), padval, jnp.int32)]
    ).reshape(NS, cap_s)
    slow = jnp.concatenate(
        [slow, jnp.full((NS, cap_f - cap_s), padval, jnp.int32)], axis=1)
    both = (fast, slow) if CF == 0 else (slow, fast)
    return jnp.stack(both, axis=1).reshape(NW, nch_f, CH)

  sg = _split(src, 0)
  sd = _split(src, N)
  dd = _split(dst, N)

  rows_acc = _rows_acc(N)
  ones8 = jnp.ones((CH, 8), jnp.float32)
  zer8 = jnp.zeros((rows_acc, 8), jnp.float32)

  degs = _make_deg(N, nch_f, nch_s, CF)(sd, dd, ones8, zer8)

  HD = W0.shape[1] // 2
  agg = _make_agg(N, nch_f, nch_s, CF, HD)
  br = 1000 if N % 1000 == 0 else N // NS

  y0A, y0B, ro, ri = _tc_first(h, W0[:, :HD], W0[:, HD:], degs, br)
  P0A = agg(y0A, sg, dd)
  P0B = agg(y0B, sg, dd)
  Wq1 = (W1[:HD, :HD], W1[HD:, :HD], W1[:HD, HD:], W1[HD:, HD:])
  y1A, y1B = _tc_mid(P0A, P0B, ri, ro, b0[:HD], b0[HD:], g0[:HD], g0[HD:],
                     be0[:HD], be0[HD:], Wq1, br, two_out=True)
  P1A = agg(y1A, sg, dd)
  P1B = agg(y1B, sg, dd)
  Wq2 = (W2[:HD], W2[HD:])
  (y2,) = _tc_mid(P1A, P1B, ri, ro, b1[:HD], b1[HD:], g1[:HD], g1[HD:],
                  be1[:HD], be1[HD:], Wq2, br, two_out=False)
  P2 = agg(y2, sg, dd)
  return _tc_last(P2, ri, b2, br)


# speedup vs baseline: 1.0167x; 1.0167x over previous
import functools

import jax
import jax.numpy as jnp
from jax import lax
from jax.experimental import pallas as pl
from jax.experimental.pallas import tpu as pltpu
from jax.experimental.pallas import tpu_sc as plsc

NC = 2
NS = 16
CH = 128
NW = NC * NS
CF = 0
FRAC = 0.73


def _rows_acc(N):
  return ((N + 1 + NS * 8 - 1) // (NS * 8)) * 8


def _make_deg(N, nch, nch_s, cf):
  rows_acc = _rows_acc(N)
  nacc = rows_acc * NS
  mesh = plsc.VectorSubcoreMesh(core_axis_name="c", subcore_axis_name="s",
                                num_cores=NC, num_subcores=NS)

  @functools.partial(
      pl.kernel, mesh=mesh,
      compiler_params=pltpu.CompilerParams(use_tc_tiling_on_sc=False),
      out_type=jax.ShapeDtypeStruct((NC, 2, nacc, 8), jnp.float32),
      scratch_types=[
          pltpu.VMEM((nch, CH), jnp.int32),
          pltpu.VMEM((nch, CH), jnp.int32),
          pltpu.VMEM((CH, 8), jnp.float32),
          pltpu.VMEM((rows_acc, 8), jnp.float32),
          pltpu.VMEM_SHARED((nacc, 8), jnp.float32),
          pltpu.VMEM_SHARED((nacc, 8), jnp.float32),
      ],
  )
  def deg(sd_hbm, dd_hbm, ones_hbm, zer_hbm, out_hbm,
          sidx, didx, obuf, zbuf, acc_s, acc_d):
    c = lax.axis_index("c")
    s = lax.axis_index("s")
    wid = s * NC + c
    my_nch = jnp.where(c == cf, nch, nch_s)
    pltpu.sync_copy(ones_hbm, obuf)
    pltpu.sync_copy(zer_hbm, zbuf)
    base = s * rows_acc
    pltpu.sync_copy(zbuf, acc_s.at[pl.ds(base, rows_acc)])
    pltpu.sync_copy(zbuf, acc_d.at[pl.ds(base, rows_acc)])
    plsc.subcore_barrier()
    pltpu.sync_copy(sd_hbm.at[wid], sidx)
    pltpu.sync_copy(dd_hbm.at[wid], didx)

    def chunk(j, _):
      pltpu.sync_copy(obuf, acc_s.at[sidx.at[j]], add=True)
      pltpu.sync_copy(obuf, acc_d.at[didx.at[j]], add=True)
      return 0

    lax.fori_loop(0, my_nch, chunk, 0)
    plsc.subcore_barrier()
    pltpu.sync_copy(acc_s.at[pl.ds(base, rows_acc)], zbuf)
    pltpu.sync_copy(zbuf, out_hbm.at[c, 0, pl.ds(base, rows_acc)])
    pltpu.sync_copy(acc_d.at[pl.ds(base, rows_acc)], zbuf)
    pltpu.sync_copy(zbuf, out_hbm.at[c, 1, pl.ds(base, rows_acc)])

  return deg


def _make_agg(N, nch, nch_s, cf, D):
  rows_acc = _rows_acc(N)
  nacc = rows_acc * NS
  mesh = plsc.VectorSubcoreMesh(core_axis_name="c", subcore_axis_name="s",
                                num_cores=NC, num_subcores=NS)

  @functools.partial(
      pl.kernel, mesh=mesh,
      compiler_params=pltpu.CompilerParams(use_tc_tiling_on_sc=False),
      out_type=jax.ShapeDtypeStruct((NC, nacc, D), jnp.float32),
      scratch_types=[
          pltpu.VMEM((nch, CH), jnp.int32),
          pltpu.VMEM((nch, CH), jnp.int32),
          pltpu.VMEM((CH, D), jnp.float32),
          pltpu.VMEM((CH, D), jnp.float32),
          pltpu.VMEM_SHARED((nacc, D), jnp.float32),
          pltpu.SemaphoreType.DMA,
          pltpu.SemaphoreType.DMA,
      ],
  )
  def agg(y_hbm, sg_hbm, dd_hbm, out_hbm,
          sidx, didx, bufa, bufb, acc, sema, semb):
    c = lax.axis_index("c")
    s = lax.axis_index("s")
    wid = s * NC + c
    my_nch = jnp.where(c == cf, nch, nch_s)
    z16 = jnp.zeros((16,), jnp.float32)

    def zrow(r, _):
      for kk in range(D // 16):
        bufa[r, pl.ds(kk * 16, 16)] = z16
      return 0

    lax.fori_loop(0, CH, zrow, 0)
    base = s * rows_acc
    nfull, rem = divmod(rows_acc, CH)
    for k in range(nfull):
      pltpu.sync_copy(bufa, acc.at[pl.ds(base + k * CH, CH)])
    if rem:
      pltpu.sync_copy(bufa.at[pl.ds(0, rem)],
                      acc.at[pl.ds(base + nfull * CH, rem)])
    plsc.subcore_barrier()
    pltpu.sync_copy(sg_hbm.at[wid], sidx)
    pltpu.sync_copy(dd_hbm.at[wid], didx)
    pltpu.async_copy(y_hbm.at[sidx.at[0]], bufa, sema)
    pltpu.async_copy(y_hbm.at[sidx.at[1]], bufb, semb)

    def pair(p, _):
      j = 2 * p
      pltpu.make_async_copy(y_hbm.at[sidx.at[j]], bufa, sema).wait()
      pltpu.sync_copy(bufa, acc.at[didx.at[j]], add=True)
      pltpu.async_copy(y_hbm.at[sidx.at[j + 2]], bufa, sema)
      pltpu.make_async_copy(y_hbm.at[sidx.at[j + 1]], bufb, semb).wait()
      pltpu.sync_copy(bufb, acc.at[didx.at[j + 1]], add=True)
      pltpu.async_copy(y_hbm.at[sidx.at[j + 3]], bufb, semb)
      return 0

    lax.fori_loop(0, my_nch // 2 - 1, pair, 0)
    j = my_nch - 2
    pltpu.make_async_copy(y_hbm.at[sidx.at[j]], bufa, sema).wait()
    pltpu.sync_copy(bufa, acc.at[didx.at[j]], add=True)
    pltpu.make_async_copy(y_hbm.at[sidx.at[j + 1]], bufb, semb).wait()
    pltpu.sync_copy(bufb, acc.at[didx.at[j + 1]], add=True)
    plsc.subcore_barrier()
    for k in range(nfull):
      pltpu.sync_copy(acc.at[pl.ds(base + k * CH, CH)], bufa)
      pltpu.sync_copy(bufa, out_hbm.at[c, pl.ds(base + k * CH, CH)])
    if rem:
      pltpu.sync_copy(acc.at[pl.ds(base + nfull * CH, rem)],
                      bufa.at[pl.ds(0, rem)])
      pltpu.sync_copy(bufa.at[pl.ds(0, rem)],
                      out_hbm.at[c, pl.ds(base + nfull * CH, rem)])

  return agg


def _tc_first(h, WA, WB, degs, br):
  N, HD = h.shape[0], WA.shape[1]

  def body(h_ref, wa_ref, wb_ref, d_ref, ya_ref, yb_ref, ro_ref, ri_ref):
    d = d_ref[...]
    dout = (d[0, 0] + d[1, 0])[:, 0:1]
    din = (d[0, 1] + d[1, 1])[:, 0:1]
    ro = lax.rsqrt(jnp.maximum(dout, 1.0))
    ri = lax.rsqrt(jnp.maximum(din, 1.0))
    x = h_ref[...]
    ya_ref[...] = jnp.dot(x, wa_ref[...], preferred_element_type=jnp.float32) * ro
    yb_ref[...] = jnp.dot(x, wb_ref[...], preferred_element_type=jnp.float32) * ro
    ro_ref[...] = ro
    ri_ref[...] = ri

  Din = h.shape[1]
  return pl.pallas_call(
      body,
      grid=(N // br,),
      in_specs=[
          pl.BlockSpec((br, Din), lambda i: (i, 0)),
          pl.BlockSpec((Din, HD), lambda i: (0, 0)),
          pl.BlockSpec((Din, HD), lambda i: (0, 0)),
          pl.BlockSpec((NC, 2, br, 8), lambda i: (0, 0, i, 0)),
      ],
      out_specs=[
          pl.BlockSpec((br, HD), lambda i: (i, 0)),
          pl.BlockSpec((br, HD), lambda i: (i, 0)),
          pl.BlockSpec((br, 1), lambda i: (i, 0)),
          pl.BlockSpec((br, 1), lambda i: (i, 0)),
      ],
      out_shape=[
          jax.ShapeDtypeStruct((N, HD), jnp.float32),
          jax.ShapeDtypeStruct((N, HD), jnp.float32),
          jax.ShapeDtypeStruct((N, 1), jnp.float32),
          jax.ShapeDtypeStruct((N, 1), jnp.float32),
      ],
  )(h, WA, WB, degs)


def _tc_mid(PA, PB, ri, ro, bA, bB, gA, gB, beA, beB, Wq, br, two_out):
  N, HD = ri.shape[0], PA.shape[2]
  Dtot = 2.0 * HD

  def body(pa_ref, pb_ref, ri_ref, ro_ref, ba_ref, bb_ref, ga_ref, gb_ref,
           bea_ref, beb_ref, *rest):
    w_refs = rest[:len(Wq)]
    y_refs = rest[len(Wq):]
    ri_v = ri_ref[...]
    xA = (pa_ref[0] + pa_ref[1]) * ri_v + ba_ref[...]
    xB = (pb_ref[0] + pb_ref[1]) * ri_v + bb_ref[...]
    mu = (jnp.sum(xA, -1, keepdims=True) + jnp.sum(xB, -1, keepdims=True)) / Dtot
    dA = xA - mu
    dB = xB - mu
    var = (jnp.sum(dA * dA, -1, keepdims=True)
           + jnp.sum(dB * dB, -1, keepdims=True)) / Dtot
    rs = lax.rsqrt(var + 1e-5)
    xA = jnp.maximum(dA * rs * ga_ref[...] + bea_ref[...], 0.0)
    xB = jnp.maximum(dB * rs * gb_ref[...] + beb_ref[...], 0.0)
    ro_v = ro_ref[...]
    y_refs[0][...] = (jnp.dot(xA, w_refs[0][...], preferred_element_type=jnp.float32)
                      + jnp.dot(xB, w_refs[1][...], preferred_element_type=jnp.float32)) * ro_v
    if two_out:
      y_refs[1][...] = (jnp.dot(xA, w_refs[2][...], preferred_element_type=jnp.float32)
                        + jnp.dot(xB, w_refs[3][...], preferred_element_type=jnp.float32)) * ro_v

  n_out = 2 if two_out else 1
  Dn = Wq[0].shape[1]
  return pl.pallas_call(
      body,
      grid=(N // br,),
      in_specs=(
          [pl.BlockSpec((NC, br, HD), lambda i: (0, i, 0))] * 2
          + [pl.BlockSpec((br, 1), lambda i: (i, 0))] * 2
          + [pl.BlockSpec((1, HD), lambda i: (0, 0))] * 6
          + [pl.BlockSpec((HD, Dn), lambda i: (0, 0))] * len(Wq)
      ),
      out_specs=[pl.BlockSpec((br, Dn), lambda i: (i, 0))] * n_out,
      out_shape=[jax.ShapeDtypeStruct((N, Dn), jnp.float32)] * n_out,
  )(PA, PB, ri, ro, bA.reshape(1, HD), bB.reshape(1, HD),
    gA.reshape(1, HD), gB.reshape(1, HD), beA.reshape(1, HD),
    beB.reshape(1, HD), *Wq)


def _tc_last(P, ri, b2, br):
  N, D = ri.shape[0], P.shape[2]

  def body(p_ref, ri_ref, b_ref, y_ref):
    y_ref[...] = (p_ref[0] + p_ref[1]) * ri_ref[...] + b_ref[...]

  return pl.pallas_call(
      body,
      grid=(N // br,),
      in_specs=[
          pl.BlockSpec((NC, br, D), lambda i: (0, i, 0)),
          pl.BlockSpec((br, 1), lambda i: (i, 0)),
          pl.BlockSpec((1, D), lambda i: (0, 0)),
      ],
      out_specs=pl.BlockSpec((br, D), lambda i: (i, 0)),
      out_shape=jax.ShapeDtypeStruct((N, D), jnp.float32),
  )(P, ri, b2.reshape(1, D))


def kernel(h, edge_index, W0, b0, g0, be0, W1, b1, g1, be1, W2, b2):
  N = h.shape[0]
  E = edge_index.shape[1]
  nch_tot = -(-E // (NS * CH)) + 1
  nch_tot += nch_tot % 2
  nch_f = int(nch_tot * FRAC) // 2 * 2
  nch_s = nch_tot - nch_f
  cap_f, cap_s = nch_f * CH, nch_s * CH
  F = NS * cap_f
  src = edge_index[0]
  dst = edge_index[1]

  def _split(a, padval):
    fast = a[:F].reshape(NS, cap_f)
    slow = jnp.concatenate(
        [a[F:], jnp.full((NS * cap_s - (E - F),), padval, jnp.int32)]
    ).reshape(NS, cap_s)
    slow = jnp.concatenate(
        [slow, jnp.full((NS, cap_f - cap_s), padval, jnp.int32)], axis=1)
    both = (fast, slow) if CF == 0 else (slow, fast)
    return jnp.stack(both, axis=1).reshape(NW, nch_f, CH)

  sg = _split(src, 0)
  sd = _split(src, N)
  dd = _split(dst, N)

  rows_acc = _rows_acc(N)
  ones8 = jnp.ones((CH, 8), jnp.float32)
  zer8 = jnp.zeros((rows_acc, 8), jnp.float32)

  degs = _make_deg(N, nch_f, nch_s, CF)(sd, dd, ones8, zer8)

  HD = W0.shape[1] // 2
  agg = _make_agg(N, nch_f, nch_s, CF, HD)
  br = 1000 if N % 1000 == 0 else N // NS

  y0A, y0B, ro, ri = _tc_first(h, W0[:, :HD], W0[:, HD:], degs, br)
  P0A = agg(y0A, sg, dd)
  P0B = agg(y0B, sg, dd)
  Wq1 = (W1[:HD, :HD], W1[HD:, :HD], W1[:HD, HD:], W1[HD:, HD:])
  y1A, y1B = _tc_mid(P0A, P0B, ri, ro, b0[:HD], b0[HD:], g0[:HD], g0[HD:],
                     be0[:HD], be0[HD:], Wq1, br, two_out=True)
  P1A = agg(y1A, sg, dd)
  P1B = agg(y1B, sg, dd)
  Wq2 = (W2[:HD], W2[HD:])
  (y2,) = _tc_mid(P1A, P1B, ri, ro, b1[:HD], b1[HD:], g1[:HD], g1[HD:],
                  be1[:HD], be1[HD:], Wq2, br, two_out=False)
  P2 = agg(y2, sg, dd)
  return _tc_last(P2, ri, b2, br)


# speedup vs baseline: 1.0207x; 1.0039x over previous
import functools

import jax
import jax.numpy as jnp
from jax import lax
from jax.experimental import pallas as pl
from jax.experimental.pallas import tpu as pltpu
from jax.experimental.pallas import tpu_sc as plsc

NC = 2
NS = 16
CH = 128
NW = NC * NS
CF = 0
FRAC = 0.73


def _rows_acc(N):
  return ((N + 1 + NS * 8 - 1) // (NS * 8)) * 8


def _make_deg(N, nch, nch_s, cf):
  rows_acc = _rows_acc(N)
  nacc = rows_acc * NS
  mesh = plsc.VectorSubcoreMesh(core_axis_name="c", subcore_axis_name="s",
                                num_cores=NC, num_subcores=NS)

  @functools.partial(
      pl.kernel, mesh=mesh,
      compiler_params=pltpu.CompilerParams(use_tc_tiling_on_sc=False),
      out_type=jax.ShapeDtypeStruct((NC, 2, nacc, 8), jnp.float32),
      scratch_types=[
          pltpu.VMEM((nch, CH), jnp.int32),
          pltpu.VMEM((nch, CH), jnp.int32),
          pltpu.VMEM((CH, 8), jnp.float32),
          pltpu.VMEM((rows_acc, 8), jnp.float32),
          pltpu.VMEM_SHARED((nacc, 8), jnp.float32),
          pltpu.VMEM_SHARED((nacc, 8), jnp.float32),
      ],
  )
  def deg(sd_hbm, dd_hbm, ones_hbm, zer_hbm, out_hbm,
          sidx, didx, obuf, zbuf, acc_s, acc_d):
    c = lax.axis_index("c")
    s = lax.axis_index("s")
    my_nch = jnp.where(c == cf, nch, nch_s)
    row_off = jnp.where(c == cf, s * nch, NS * nch + s * nch_s)
    pltpu.sync_copy(ones_hbm, obuf)
    pltpu.sync_copy(zer_hbm, zbuf)
    base = s * rows_acc
    pltpu.sync_copy(zbuf, acc_s.at[pl.ds(base, rows_acc)])
    pltpu.sync_copy(zbuf, acc_d.at[pl.ds(base, rows_acc)])
    plsc.subcore_barrier()
    pltpu.sync_copy(sd_hbm.at[pl.ds(row_off, nch)], sidx)
    pltpu.sync_copy(dd_hbm.at[pl.ds(row_off, nch)], didx)

    def chunk(j, _):
      pltpu.sync_copy(obuf, acc_s.at[sidx.at[j]], add=True)
      pltpu.sync_copy(obuf, acc_d.at[didx.at[j]], add=True)
      return 0

    lax.fori_loop(0, my_nch, chunk, 0)
    plsc.subcore_barrier()
    pltpu.sync_copy(acc_s.at[pl.ds(base, rows_acc)], zbuf)
    pltpu.sync_copy(zbuf, out_hbm.at[c, 0, pl.ds(base, rows_acc)])
    pltpu.sync_copy(acc_d.at[pl.ds(base, rows_acc)], zbuf)
    pltpu.sync_copy(zbuf, out_hbm.at[c, 1, pl.ds(base, rows_acc)])

  return deg


def _make_agg(N, nch, nch_s, cf, D):
  rows_acc = _rows_acc(N)
  nacc = rows_acc * NS
  mesh = plsc.VectorSubcoreMesh(core_axis_name="c", subcore_axis_name="s",
                                num_cores=NC, num_subcores=NS)

  @functools.partial(
      pl.kernel, mesh=mesh,
      compiler_params=pltpu.CompilerParams(use_tc_tiling_on_sc=False),
      out_type=jax.ShapeDtypeStruct((NC, nacc, D), jnp.float32),
      scratch_types=[
          pltpu.VMEM((nch, CH), jnp.int32),
          pltpu.VMEM((nch, CH), jnp.int32),
          pltpu.VMEM((CH, D), jnp.float32),
          pltpu.VMEM((CH, D), jnp.float32),
          pltpu.VMEM_SHARED((nacc, D), jnp.float32),
          pltpu.SemaphoreType.DMA,
          pltpu.SemaphoreType.DMA,
      ],
  )
  def agg(y_hbm, sg_hbm, dd_hbm, out_hbm,
          sidx, didx, bufa, bufb, acc, sema, semb):
    c = lax.axis_index("c")
    s = lax.axis_index("s")
    my_nch = jnp.where(c == cf, nch, nch_s)
    row_off = jnp.where(c == cf, s * nch, NS * nch + s * nch_s)
    z16 = jnp.zeros((16,), jnp.float32)

    def zrow(r, _):
      for kk in range(D // 16):
        bufa[r, pl.ds(kk * 16, 16)] = z16
      return 0

    lax.fori_loop(0, CH, zrow, 0)
    base = s * rows_acc
    nfull, rem = divmod(rows_acc, CH)
    for k in range(nfull):
      pltpu.sync_copy(bufa, acc.at[pl.ds(base + k * CH, CH)])
    if rem:
      pltpu.sync_copy(bufa.at[pl.ds(0, rem)],
                      acc.at[pl.ds(base + nfull * CH, rem)])
    plsc.subcore_barrier()
    pltpu.sync_copy(sg_hbm.at[pl.ds(row_off, nch)], sidx)
    pltpu.sync_copy(dd_hbm.at[pl.ds(row_off, nch)], didx)
    nclamp = jnp.full((16,), N - 1, jnp.int32)

    def crow(r, _):
      for kk in range(CH // 16):
        sidx[r, pl.ds(kk * 16, 16)] = jnp.minimum(
            sidx[r, pl.ds(kk * 16, 16)], nclamp)
      return 0

    lax.fori_loop(0, nch, crow, 0)
    pltpu.async_copy(y_hbm.at[sidx.at[0]], bufa, sema)
    pltpu.async_copy(y_hbm.at[sidx.at[1]], bufb, semb)

    def pair(p, _):
      j = 2 * p
      pltpu.make_async_copy(y_hbm.at[sidx.at[j]], bufa, sema).wait()
      pltpu.sync_copy(bufa, acc.at[didx.at[j]], add=True)
      pltpu.async_copy(y_hbm.at[sidx.at[j + 2]], bufa, sema)
      pltpu.make_async_copy(y_hbm.at[sidx.at[j + 1]], bufb, semb).wait()
      pltpu.sync_copy(bufb, acc.at[didx.at[j + 1]], add=True)
      pltpu.async_copy(y_hbm.at[sidx.at[j + 3]], bufb, semb)
      return 0

    lax.fori_loop(0, my_nch // 2 - 1, pair, 0)
    j = my_nch - 2
    pltpu.make_async_copy(y_hbm.at[sidx.at[j]], bufa, sema).wait()
    pltpu.sync_copy(bufa, acc.at[didx.at[j]], add=True)
    pltpu.make_async_copy(y_hbm.at[sidx.at[j + 1]], bufb, semb).wait()
    pltpu.sync_copy(bufb, acc.at[didx.at[j + 1]], add=True)
    plsc.subcore_barrier()
    for k in range(nfull):
      pltpu.sync_copy(acc.at[pl.ds(base + k * CH, CH)], bufa)
      pltpu.sync_copy(bufa, out_hbm.at[c, pl.ds(base + k * CH, CH)])
    if rem:
      pltpu.sync_copy(acc.at[pl.ds(base + nfull * CH, rem)],
                      bufa.at[pl.ds(0, rem)])
      pltpu.sync_copy(bufa.at[pl.ds(0, rem)],
                      out_hbm.at[c, pl.ds(base + nfull * CH, rem)])

  return agg


def _tc_first(h, WA, WB, degs, br):
  N, HD = h.shape[0], WA.shape[1]

  def body(h_ref, wa_ref, wb_ref, d_ref, ya_ref, yb_ref, ro_ref, ri_ref):
    d = d_ref[...]
    dout = (d[0, 0] + d[1, 0])[:, 0:1]
    din = (d[0, 1] + d[1, 1])[:, 0:1]
    ro = lax.rsqrt(jnp.maximum(dout, 1.0))
    ri = lax.rsqrt(jnp.maximum(din, 1.0))
    x = h_ref[...]
    ya_ref[...] = jnp.dot(x, wa_ref[...], preferred_element_type=jnp.float32) * ro
    yb_ref[...] = jnp.dot(x, wb_ref[...], preferred_element_type=jnp.float32) * ro
    ro_ref[...] = ro
    ri_ref[...] = ri

  Din = h.shape[1]
  return pl.pallas_call(
      body,
      grid=(N // br,),
      in_specs=[
          pl.BlockSpec((br, Din), lambda i: (i, 0)),
          pl.BlockSpec((Din, HD), lambda i: (0, 0)),
          pl.BlockSpec((Din, HD), lambda i: (0, 0)),
          pl.BlockSpec((NC, 2, br, 8), lambda i: (0, 0, i, 0)),
      ],
      out_specs=[
          pl.BlockSpec((br, HD), lambda i: (i, 0)),
          pl.BlockSpec((br, HD), lambda i: (i, 0)),
          pl.BlockSpec((br, 1), lambda i: (i, 0)),
          pl.BlockSpec((br, 1), lambda i: (i, 0)),
      ],
      out_shape=[
          jax.ShapeDtypeStruct((N, HD), jnp.float32),
          jax.ShapeDtypeStruct((N, HD), jnp.float32),
          jax.ShapeDtypeStruct((N, 1), jnp.float32),
          jax.ShapeDtypeStruct((N, 1), jnp.float32),
      ],
  )(h, WA, WB, degs)


def _tc_mid(PA, PB, ri, ro, bA, bB, gA, gB, beA, beB, Wq, br, two_out):
  N, HD = ri.shape[0], PA.shape[2]
  Dtot = 2.0 * HD

  def body(pa_ref, pb_ref, ri_ref, ro_ref, ba_ref, bb_ref, ga_ref, gb_ref,
           bea_ref, beb_ref, *rest):
    w_refs = rest[:len(Wq)]
    y_refs = rest[len(Wq):]
    ri_v = ri_ref[...]
    xA = (pa_ref[0] + pa_ref[1]) * ri_v + ba_ref[...]
    xB = (pb_ref[0] + pb_ref[1]) * ri_v + bb_ref[...]
    mu = (jnp.sum(xA, -1, keepdims=True) + jnp.sum(xB, -1, keepdims=True)) / Dtot
    dA = xA - mu
    dB = xB - mu
    var = (jnp.sum(dA * dA, -1, keepdims=True)
           + jnp.sum(dB * dB, -1, keepdims=True)) / Dtot
    rs = lax.rsqrt(var + 1e-5)
    xA = jnp.maximum(dA * rs * ga_ref[...] + bea_ref[...], 0.0)
    xB = jnp.maximum(dB * rs * gb_ref[...] + beb_ref[...], 0.0)
    ro_v = ro_ref[...]
    y_refs[0][...] = (jnp.dot(xA, w_refs[0][...], preferred_element_type=jnp.float32)
                      + jnp.dot(xB, w_refs[1][...], preferred_element_type=jnp.float32)) * ro_v
    if two_out:
      y_refs[1][...] = (jnp.dot(xA, w_refs[2][...], preferred_element_type=jnp.float32)
                        + jnp.dot(xB, w_refs[3][...], preferred_element_type=jnp.float32)) * ro_v

  n_out = 2 if two_out else 1
  Dn = Wq[0].shape[1]
  return pl.pallas_call(
      body,
      grid=(N // br,),
      in_specs=(
          [pl.BlockSpec((NC, br, HD), lambda i: (0, i, 0))] * 2
          + [pl.BlockSpec((br, 1), lambda i: (i, 0))] * 2
          + [pl.BlockSpec((1, HD), lambda i: (0, 0))] * 6
          + [pl.BlockSpec((HD, Dn), lambda i: (0, 0))] * len(Wq)
      ),
      out_specs=[pl.BlockSpec((br, Dn), lambda i: (i, 0))] * n_out,
      out_shape=[jax.ShapeDtypeStruct((N, Dn), jnp.float32)] * n_out,
  )(PA, PB, ri, ro, bA.reshape(1, HD), bB.reshape(1, HD),
    gA.reshape(1, HD), gB.reshape(1, HD), beA.reshape(1, HD),
    beB.reshape(1, HD), *Wq)


def _tc_last(P, ri, b2, br):
  N, D = ri.shape[0], P.shape[2]

  def body(p_ref, ri_ref, b_ref, y_ref):
    y_ref[...] = (p_ref[0] + p_ref[1]) * ri_ref[...] + b_ref[...]

  return pl.pallas_call(
      body,
      grid=(N // br,),
      in_specs=[
          pl.BlockSpec((NC, br, D), lambda i: (0, i, 0)),
          pl.BlockSpec((br, 1), lambda i: (i, 0)),
          pl.BlockSpec((1, D), lambda i: (0, 0)),
      ],
      out_specs=pl.BlockSpec((br, D), lambda i: (i, 0)),
      out_shape=jax.ShapeDtypeStruct((N, D), jnp.float32),
  )(P, ri, b2.reshape(1, D))


def kernel(h, edge_index, W0, b0, g0, be0, W1, b1, g1, be1, W2, b2):
  N = h.shape[0]
  E = edge_index.shape[1]
  nch_tot = -(-E // (NS * CH)) + 1
  nch_tot += nch_tot % 2
  nch_f = int(nch_tot * FRAC) // 2 * 2
  nch_s = nch_tot - nch_f
  cap_f, cap_s = nch_f * CH, nch_s * CH
  F = NS * cap_f
  src = edge_index[0]
  dst = edge_index[1]

  R = NS * nch_f + (NS - 1) * nch_s + nch_f
  pad = R * CH - E
  sd = jnp.concatenate([src, jnp.full((pad,), N, jnp.int32)]).reshape(R, CH)
  dd = jnp.concatenate([dst, jnp.full((pad,), N, jnp.int32)]).reshape(R, CH)
  sg = sd

  rows_acc = _rows_acc(N)
  ones8 = jnp.ones((CH, 8), jnp.float32)
  zer8 = jnp.zeros((rows_acc, 8), jnp.float32)

  degs = _make_deg(N, nch_f, nch_s, CF)(sd, dd, ones8, zer8)

  HD = W0.shape[1] // 2
  agg = _make_agg(N, nch_f, nch_s, CF, HD)
  br = 1000 if N % 1000 == 0 else N // NS

  y0A, y0B, ro, ri = _tc_first(h, W0[:, :HD], W0[:, HD:], degs, br)
  P0A = agg(y0A, sg, dd)
  P0B = agg(y0B, sg, dd)
  Wq1 = (W1[:HD, :HD], W1[HD:, :HD], W1[:HD, HD:], W1[HD:, HD:])
  y1A, y1B = _tc_mid(P0A, P0B, ri, ro, b0[:HD], b0[HD:], g0[:HD], g0[HD:],
                     be0[:HD], be0[HD:], Wq1, br, two_out=True)
  P1A = agg(y1A, sg, dd)
  P1B = agg(y1B, sg, dd)
  Wq2 = (W2[:HD], W2[HD:])
  (y2,) = _tc_mid(P1A, P1B, ri, ro, b1[:HD], b1[HD:], g1[:HD], g1[HD:],
                  be1[:HD], be1[HD:], Wq2, br, two_out=False)
  P2 = agg(y2, sg, dd)
  return _tc_last(P2, ri, b2, br)


# speedup vs baseline: 1.0269x; 1.0061x over previous
import functools

import jax
import jax.numpy as jnp
from jax import lax
from jax.experimental import pallas as pl
from jax.experimental.pallas import tpu as pltpu
from jax.experimental.pallas import tpu_sc as plsc

NC = 2
NS = 16
CH = 128
NW = NC * NS
CF = 0
FRAC = 0.75


def _rows_acc(N):
  return ((N + 1 + NS * 8 - 1) // (NS * 8)) * 8


def _make_deg(N, nch, nch_s, cf):
  rows_acc = _rows_acc(N)
  nacc = rows_acc * NS
  mesh = plsc.VectorSubcoreMesh(core_axis_name="c", subcore_axis_name="s",
                                num_cores=NC, num_subcores=NS)

  @functools.partial(
      pl.kernel, mesh=mesh,
      compiler_params=pltpu.CompilerParams(use_tc_tiling_on_sc=False),
      out_type=jax.ShapeDtypeStruct((NC, 2, nacc, 8), jnp.float32),
      scratch_types=[
          pltpu.VMEM((nch, CH), jnp.int32),
          pltpu.VMEM((nch, CH), jnp.int32),
          pltpu.VMEM((CH, 8), jnp.float32),
          pltpu.VMEM((rows_acc, 8), jnp.float32),
          pltpu.VMEM_SHARED((nacc, 8), jnp.float32),
          pltpu.VMEM_SHARED((nacc, 8), jnp.float32),
      ],
  )
  def deg(sd_hbm, dd_hbm, ones_hbm, zer_hbm, out_hbm,
          sidx, didx, obuf, zbuf, acc_s, acc_d):
    c = lax.axis_index("c")
    s = lax.axis_index("s")
    my_nch = jnp.where(c == cf, nch, nch_s)
    row_off = jnp.where(c == cf, s * nch, NS * nch + s * nch_s)
    pltpu.sync_copy(ones_hbm, obuf)
    pltpu.sync_copy(zer_hbm, zbuf)
    base = s * rows_acc
    pltpu.sync_copy(zbuf, acc_s.at[pl.ds(base, rows_acc)])
    pltpu.sync_copy(zbuf, acc_d.at[pl.ds(base, rows_acc)])
    plsc.subcore_barrier()
    pltpu.sync_copy(sd_hbm.at[pl.ds(row_off, nch)], sidx)
    pltpu.sync_copy(dd_hbm.at[pl.ds(row_off, nch)], didx)

    def chunk(j, _):
      pltpu.sync_copy(obuf, acc_s.at[sidx.at[j]], add=True)
      pltpu.sync_copy(obuf, acc_d.at[didx.at[j]], add=True)
      return 0

    lax.fori_loop(0, my_nch, chunk, 0)
    plsc.subcore_barrier()
    pltpu.sync_copy(acc_s.at[pl.ds(base, rows_acc)], zbuf)
    pltpu.sync_copy(zbuf, out_hbm.at[c, 0, pl.ds(base, rows_acc)])
    pltpu.sync_copy(acc_d.at[pl.ds(base, rows_acc)], zbuf)
    pltpu.sync_copy(zbuf, out_hbm.at[c, 1, pl.ds(base, rows_acc)])

  return deg


def _make_agg(N, nch, nch_s, cf, D):
  rows_acc = _rows_acc(N)
  nacc = rows_acc * NS
  mesh = plsc.VectorSubcoreMesh(core_axis_name="c", subcore_axis_name="s",
                                num_cores=NC, num_subcores=NS)

  @functools.partial(
      pl.kernel, mesh=mesh,
      compiler_params=pltpu.CompilerParams(use_tc_tiling_on_sc=False),
      out_type=jax.ShapeDtypeStruct((NC, nacc, D), jnp.float32),
      scratch_types=[
          pltpu.VMEM((nch, CH), jnp.int32),
          pltpu.VMEM((nch, CH), jnp.int32),
          pltpu.VMEM((CH, D), jnp.float32),
          pltpu.VMEM((CH, D), jnp.float32),
          pltpu.VMEM_SHARED((nacc, D), jnp.float32),
          pltpu.SemaphoreType.DMA,
          pltpu.SemaphoreType.DMA,
      ],
  )
  def agg(y_hbm, sg_hbm, dd_hbm, out_hbm,
          sidx, didx, bufa, bufb, acc, sema, semb):
    c = lax.axis_index("c")
    s = lax.axis_index("s")
    my_nch = jnp.where(c == cf, nch, nch_s)
    row_off = jnp.where(c == cf, s * nch, NS * nch + s * nch_s)
    z16 = jnp.zeros((16,), jnp.float32)

    def zrow(r, _):
      for kk in range(D // 16):
        bufa[r, pl.ds(kk * 16, 16)] = z16
      return 0

    lax.fori_loop(0, CH, zrow, 0)
    base = s * rows_acc
    nfull, rem = divmod(rows_acc, CH)
    for k in range(nfull):
      pltpu.sync_copy(bufa, acc.at[pl.ds(base + k * CH, CH)])
    if rem:
      pltpu.sync_copy(bufa.at[pl.ds(0, rem)],
                      acc.at[pl.ds(base + nfull * CH, rem)])
    plsc.subcore_barrier()
    pltpu.sync_copy(sg_hbm.at[pl.ds(row_off, nch)], sidx)
    pltpu.sync_copy(dd_hbm.at[pl.ds(row_off, nch)], didx)
    nclamp = jnp.full((16,), N - 1, jnp.int32)

    def crow(r, _):
      for kk in range(CH // 16):
        sidx[r, pl.ds(kk * 16, 16)] = jnp.minimum(
            sidx[r, pl.ds(kk * 16, 16)], nclamp)
      return 0

    lax.fori_loop(0, nch, crow, 0)
    pltpu.async_copy(y_hbm.at[sidx.at[0]], bufa, sema)
    pltpu.async_copy(y_hbm.at[sidx.at[1]], bufb, semb)

    def pair(p, _):
      j = 2 * p
      pltpu.make_async_copy(y_hbm.at[sidx.at[j]], bufa, sema).wait()
      pltpu.sync_copy(bufa, acc.at[didx.at[j]], add=True)
      pltpu.async_copy(y_hbm.at[sidx.at[j + 2]], bufa, sema)
      pltpu.make_async_copy(y_hbm.at[sidx.at[j + 1]], bufb, semb).wait()
      pltpu.sync_copy(bufb, acc.at[didx.at[j + 1]], add=True)
      pltpu.async_copy(y_hbm.at[sidx.at[j + 3]], bufb, semb)
      return 0

    lax.fori_loop(0, my_nch // 2 - 1, pair, 0)
    j = my_nch - 2
    pltpu.make_async_copy(y_hbm.at[sidx.at[j]], bufa, sema).wait()
    pltpu.sync_copy(bufa, acc.at[didx.at[j]], add=True)
    pltpu.make_async_copy(y_hbm.at[sidx.at[j + 1]], bufb, semb).wait()
    pltpu.sync_copy(bufb, acc.at[didx.at[j + 1]], add=True)
    plsc.subcore_barrier()
    for k in range(nfull):
      pltpu.sync_copy(acc.at[pl.ds(base + k * CH, CH)], bufa)
      pltpu.sync_copy(bufa, out_hbm.at[c, pl.ds(base + k * CH, CH)])
    if rem:
      pltpu.sync_copy(acc.at[pl.ds(base + nfull * CH, rem)],
                      bufa.at[pl.ds(0, rem)])
      pltpu.sync_copy(bufa.at[pl.ds(0, rem)],
                      out_hbm.at[c, pl.ds(base + nfull * CH, rem)])

  return agg


def _tc_first(h, WA, WB, degs, br):
  N, HD = h.shape[0], WA.shape[1]

  def body(h_ref, wa_ref, wb_ref, d_ref, ya_ref, yb_ref, ro_ref, ri_ref):
    d = d_ref[...]
    dout = (d[0, 0] + d[1, 0])[:, 0:1]
    din = (d[0, 1] + d[1, 1])[:, 0:1]
    ro = lax.rsqrt(jnp.maximum(dout, 1.0))
    ri = lax.rsqrt(jnp.maximum(din, 1.0))
    x = h_ref[...]
    ya_ref[...] = jnp.dot(x, wa_ref[...], preferred_element_type=jnp.float32) * ro
    yb_ref[...] = jnp.dot(x, wb_ref[...], preferred_element_type=jnp.float32) * ro
    ro_ref[...] = ro
    ri_ref[...] = ri

  Din = h.shape[1]
  return pl.pallas_call(
      body,
      grid=(N // br,),
      in_specs=[
          pl.BlockSpec((br, Din), lambda i: (i, 0)),
          pl.BlockSpec((Din, HD), lambda i: (0, 0)),
          pl.BlockSpec((Din, HD), lambda i: (0, 0)),
          pl.BlockSpec((NC, 2, br, 8), lambda i: (0, 0, i, 0)),
      ],
      out_specs=[
          pl.BlockSpec((br, HD), lambda i: (i, 0)),
          pl.BlockSpec((br, HD), lambda i: (i, 0)),
          pl.BlockSpec((br, 1), lambda i: (i, 0)),
          pl.BlockSpec((br, 1), lambda i: (i, 0)),
      ],
      out_shape=[
          jax.ShapeDtypeStruct((N, HD), jnp.float32),
          jax.ShapeDtypeStruct((N, HD), jnp.float32),
          jax.ShapeDtypeStruct((N, 1), jnp.float32),
          jax.ShapeDtypeStruct((N, 1), jnp.float32),
      ],
  )(h, WA, WB, degs)


def _tc_mid(PA, PB, ri, ro, bA, bB, gA, gB, beA, beB, Wq, br, two_out):
  N, HD = ri.shape[0], PA.shape[2]
  Dtot = 2.0 * HD

  def body(pa_ref, pb_ref, ri_ref, ro_ref, ba_ref, bb_ref, ga_ref, gb_ref,
           bea_ref, beb_ref, *rest):
    w_refs = rest[:len(Wq)]
    y_refs = rest[len(Wq):]
    ri_v = ri_ref[...]
    xA = (pa_ref[0] + pa_ref[1]) * ri_v + ba_ref[...]
    xB = (pb_ref[0] + pb_ref[1]) * ri_v + bb_ref[...]
    mu = (jnp.sum(xA, -1, keepdims=True) + jnp.sum(xB, -1, keepdims=True)) / Dtot
    dA = xA - mu
    dB = xB - mu
    var = (jnp.sum(dA * dA, -1, keepdims=True)
           + jnp.sum(dB * dB, -1, keepdims=True)) / Dtot
    rs = lax.rsqrt(var + 1e-5)
    xA = jnp.maximum(dA * rs * ga_ref[...] + bea_ref[...], 0.0)
    xB = jnp.maximum(dB * rs * gb_ref[...] + beb_ref[...], 0.0)
    ro_v = ro_ref[...]
    y_refs[0][...] = (jnp.dot(xA, w_refs[0][...], preferred_element_type=jnp.float32)
                      + jnp.dot(xB, w_refs[1][...], preferred_element_type=jnp.float32)) * ro_v
    if two_out:
      y_refs[1][...] = (jnp.dot(xA, w_refs[2][...], preferred_element_type=jnp.float32)
                        + jnp.dot(xB, w_refs[3][...], preferred_element_type=jnp.float32)) * ro_v

  n_out = 2 if two_out else 1
  Dn = Wq[0].shape[1]
  return pl.pallas_call(
      body,
      grid=(N // br,),
      in_specs=(
          [pl.BlockSpec((NC, br, HD), lambda i: (0, i, 0))] * 2
          + [pl.BlockSpec((br, 1), lambda i: (i, 0))] * 2
          + [pl.BlockSpec((1, HD), lambda i: (0, 0))] * 6
          + [pl.BlockSpec((HD, Dn), lambda i: (0, 0))] * len(Wq)
      ),
      out_specs=[pl.BlockSpec((br, Dn), lambda i: (i, 0))] * n_out,
      out_shape=[jax.ShapeDtypeStruct((N, Dn), jnp.float32)] * n_out,
  )(PA, PB, ri, ro, bA.reshape(1, HD), bB.reshape(1, HD),
    gA.reshape(1, HD), gB.reshape(1, HD), beA.reshape(1, HD),
    beB.reshape(1, HD), *Wq)


def _tc_last(P, ri, b2, br):
  N, D = ri.shape[0], P.shape[2]

  def body(p_ref, ri_ref, b_ref, y_ref):
    y_ref[...] = (p_ref[0] + p_ref[1]) * ri_ref[...] + b_ref[...]

  return pl.pallas_call(
      body,
      grid=(N // br,),
      in_specs=[
          pl.BlockSpec((NC, br, D), lambda i: (0, i, 0)),
          pl.BlockSpec((br, 1), lambda i: (i, 0)),
          pl.BlockSpec((1, D), lambda i: (0, 0)),
      ],
      out_specs=pl.BlockSpec((br, D), lambda i: (i, 0)),
      out_shape=jax.ShapeDtypeStruct((N, D), jnp.float32),
  )(P, ri, b2.reshape(1, D))


def kernel(h, edge_index, W0, b0, g0, be0, W1, b1, g1, be1, W2, b2):
  N = h.shape[0]
  E = edge_index.shape[1]
  nch_tot = -(-E // (NS * CH)) + 1
  nch_tot += nch_tot % 2
  nch_f = int(nch_tot * FRAC) // 2 * 2
  nch_s = nch_tot - nch_f
  cap_f, cap_s = nch_f * CH, nch_s * CH
  F = NS * cap_f
  src = edge_index[0]
  dst = edge_index[1]

  R = NS * nch_f + (NS - 1) * nch_s + nch_f
  pad = R * CH - E
  sd = jnp.concatenate([src, jnp.full((pad,), N, jnp.int32)]).reshape(R, CH)
  dd = jnp.concatenate([dst, jnp.full((pad,), N, jnp.int32)]).reshape(R, CH)
  sg = sd

  rows_acc = _rows_acc(N)
  ones8 = jnp.ones((CH, 8), jnp.float32)
  zer8 = jnp.zeros((rows_acc, 8), jnp.float32)

  degs = _make_deg(N, nch_f, nch_s, CF)(sd, dd, ones8, zer8)

  HD = W0.shape[1] // 2
  agg = _make_agg(N, nch_f, nch_s, CF, HD)
  br = 1000 if N % 1000 == 0 else N // NS

  y0A, y0B, ro, ri = _tc_first(h, W0[:, :HD], W0[:, HD:], degs, br)
  P0A = agg(y0A, sg, dd)
  P0B = agg(y0B, sg, dd)
  Wq1 = (W1[:HD, :HD], W1[HD:, :HD], W1[:HD, HD:], W1[HD:, HD:])
  y1A, y1B = _tc_mid(P0A, P0B, ri, ro, b0[:HD], b0[HD:], g0[:HD], g0[HD:],
                     be0[:HD], be0[HD:], Wq1, br, two_out=True)
  P1A = agg(y1A, sg, dd)
  P1B = agg(y1B, sg, dd)
  Wq2 = (W2[:HD], W2[HD:])
  (y2,) = _tc_mid(P1A, P1B, ri, ro, b1[:HD], b1[HD:], g1[:HD], g1[HD:],
                  be1[:HD], be1[HD:], Wq2, br, two_out=False)
  P2 = agg(y2, sg, dd)
  return _tc_last(P2, ri, b2, br)


# speedup vs baseline: 1.0293x; 1.0023x over previous
import functools

import jax
import jax.numpy as jnp
from jax import lax
from jax.experimental import pallas as pl
from jax.experimental.pallas import tpu as pltpu
from jax.experimental.pallas import tpu_sc as plsc

NC = 2
NS = 16
CH = 128
NW = NC * NS
CF = 0
FRAC = 0.77


def _rows_acc(N):
  return ((N + 1 + NS * 8 - 1) // (NS * 8)) * 8


def _make_deg(N, nch, nch_s, cf):
  rows_acc = _rows_acc(N)
  nacc = rows_acc * NS
  mesh = plsc.VectorSubcoreMesh(core_axis_name="c", subcore_axis_name="s",
                                num_cores=NC, num_subcores=NS)

  @functools.partial(
      pl.kernel, mesh=mesh,
      compiler_params=pltpu.CompilerParams(use_tc_tiling_on_sc=False),
      out_type=jax.ShapeDtypeStruct((NC, 2, nacc, 8), jnp.float32),
      scratch_types=[
          pltpu.VMEM((nch, CH), jnp.int32),
          pltpu.VMEM((nch, CH), jnp.int32),
          pltpu.VMEM((CH, 8), jnp.float32),
          pltpu.VMEM((rows_acc, 8), jnp.float32),
          pltpu.VMEM_SHARED((nacc, 8), jnp.float32),
          pltpu.VMEM_SHARED((nacc, 8), jnp.float32),
      ],
  )
  def deg(sd_hbm, dd_hbm, ones_hbm, zer_hbm, out_hbm,
          sidx, didx, obuf, zbuf, acc_s, acc_d):
    c = lax.axis_index("c")
    s = lax.axis_index("s")
    my_nch = jnp.where(c == cf, nch, nch_s)
    row_off = jnp.where(c == cf, s * nch, NS * nch + s * nch_s)
    pltpu.sync_copy(ones_hbm, obuf)
    pltpu.sync_copy(zer_hbm, zbuf)
    base = s * rows_acc
    pltpu.sync_copy(zbuf, acc_s.at[pl.ds(base, rows_acc)])
    pltpu.sync_copy(zbuf, acc_d.at[pl.ds(base, rows_acc)])
    plsc.subcore_barrier()
    pltpu.sync_copy(sd_hbm.at[pl.ds(row_off, nch)], sidx)
    pltpu.sync_copy(dd_hbm.at[pl.ds(row_off, nch)], didx)

    def chunk(j, _):
      pltpu.sync_copy(obuf, acc_s.at[sidx.at[j]], add=True)
      pltpu.sync_copy(obuf, acc_d.at[didx.at[j]], add=True)
      return 0

    lax.fori_loop(0, my_nch, chunk, 0)
    plsc.subcore_barrier()
    pltpu.sync_copy(acc_s.at[pl.ds(base, rows_acc)], zbuf)
    pltpu.sync_copy(zbuf, out_hbm.at[c, 0, pl.ds(base, rows_acc)])
    pltpu.sync_copy(acc_d.at[pl.ds(base, rows_acc)], zbuf)
    pltpu.sync_copy(zbuf, out_hbm.at[c, 1, pl.ds(base, rows_acc)])

  return deg


def _make_agg(N, nch, nch_s, cf, D):
  rows_acc = _rows_acc(N)
  nacc = rows_acc * NS
  mesh = plsc.VectorSubcoreMesh(core_axis_name="c", subcore_axis_name="s",
                                num_cores=NC, num_subcores=NS)

  @functools.partial(
      pl.kernel, mesh=mesh,
      compiler_params=pltpu.CompilerParams(use_tc_tiling_on_sc=False),
      out_type=jax.ShapeDtypeStruct((NC, nacc, D), jnp.float32),
      scratch_types=[
          pltpu.VMEM((nch, CH), jnp.int32),
          pltpu.VMEM((nch, CH), jnp.int32),
          pltpu.VMEM((CH, D), jnp.float32),
          pltpu.VMEM((CH, D), jnp.float32),
          pltpu.VMEM_SHARED((nacc, D), jnp.float32),
          pltpu.SemaphoreType.DMA,
          pltpu.SemaphoreType.DMA,
      ],
  )
  def agg(y_hbm, sg_hbm, dd_hbm, out_hbm,
          sidx, didx, bufa, bufb, acc, sema, semb):
    c = lax.axis_index("c")
    s = lax.axis_index("s")
    my_nch = jnp.where(c == cf, nch, nch_s)
    row_off = jnp.where(c == cf, s * nch, NS * nch + s * nch_s)
    z16 = jnp.zeros((16,), jnp.float32)

    def zrow(r, _):
      for kk in range(D // 16):
        bufa[r, pl.ds(kk * 16, 16)] = z16
      return 0

    lax.fori_loop(0, CH, zrow, 0)
    base = s * rows_acc
    nfull, rem = divmod(rows_acc, CH)
    for k in range(nfull):
      pltpu.sync_copy(bufa, acc.at[pl.ds(base + k * CH, CH)])
    if rem:
      pltpu.sync_copy(bufa.at[pl.ds(0, rem)],
                      acc.at[pl.ds(base + nfull * CH, rem)])
    plsc.subcore_barrier()
    pltpu.sync_copy(sg_hbm.at[pl.ds(row_off, nch)], sidx)
    pltpu.sync_copy(dd_hbm.at[pl.ds(row_off, nch)], didx)
    nclamp = jnp.full((16,), N - 1, jnp.int32)

    def crow(r, _):
      for kk in range(CH // 16):
        sidx[r, pl.ds(kk * 16, 16)] = jnp.minimum(
            sidx[r, pl.ds(kk * 16, 16)], nclamp)
      return 0

    lax.fori_loop(0, nch, crow, 0)
    pltpu.async_copy(y_hbm.at[sidx.at[0]], bufa, sema)
    pltpu.async_copy(y_hbm.at[sidx.at[1]], bufb, semb)

    def pair(p, _):
      j = 2 * p
      pltpu.make_async_copy(y_hbm.at[sidx.at[j]], bufa, sema).wait()
      pltpu.sync_copy(bufa, acc.at[didx.at[j]], add=True)
      pltpu.async_copy(y_hbm.at[sidx.at[j + 2]], bufa, sema)
      pltpu.make_async_copy(y_hbm.at[sidx.at[j + 1]], bufb, semb).wait()
      pltpu.sync_copy(bufb, acc.at[didx.at[j + 1]], add=True)
      pltpu.async_copy(y_hbm.at[sidx.at[j + 3]], bufb, semb)
      return 0

    lax.fori_loop(0, my_nch // 2 - 1, pair, 0)
    j = my_nch - 2
    pltpu.make_async_copy(y_hbm.at[sidx.at[j]], bufa, sema).wait()
    pltpu.sync_copy(bufa, acc.at[didx.at[j]], add=True)
    pltpu.make_async_copy(y_hbm.at[sidx.at[j + 1]], bufb, semb).wait()
    pltpu.sync_copy(bufb, acc.at[didx.at[j + 1]], add=True)
    plsc.subcore_barrier()
    for k in range(nfull):
      pltpu.sync_copy(acc.at[pl.ds(base + k * CH, CH)], bufa)
      pltpu.sync_copy(bufa, out_hbm.at[c, pl.ds(base + k * CH, CH)])
    if rem:
      pltpu.sync_copy(acc.at[pl.ds(base + nfull * CH, rem)],
                      bufa.at[pl.ds(0, rem)])
      pltpu.sync_copy(bufa.at[pl.ds(0, rem)],
                      out_hbm.at[c, pl.ds(base + nfull * CH, rem)])

  return agg


def _tc_first(h, WA, WB, degs, br):
  N, HD = h.shape[0], WA.shape[1]

  def body(h_ref, wa_ref, wb_ref, d_ref, ya_ref, yb_ref, ro_ref, ri_ref):
    d = d_ref[...]
    dout = (d[0, 0] + d[1, 0])[:, 0:1]
    din = (d[0, 1] + d[1, 1])[:, 0:1]
    ro = lax.rsqrt(jnp.maximum(dout, 1.0))
    ri = lax.rsqrt(jnp.maximum(din, 1.0))
    x = h_ref[...]
    ya_ref[...] = jnp.dot(x, wa_ref[...], preferred_element_type=jnp.float32) * ro
    yb_ref[...] = jnp.dot(x, wb_ref[...], preferred_element_type=jnp.float32) * ro
    ro_ref[...] = ro
    ri_ref[...] = ri

  Din = h.shape[1]
  return pl.pallas_call(
      body,
      grid=(N // br,),
      in_specs=[
          pl.BlockSpec((br, Din), lambda i: (i, 0)),
          pl.BlockSpec((Din, HD), lambda i: (0, 0)),
          pl.BlockSpec((Din, HD), lambda i: (0, 0)),
          pl.BlockSpec((NC, 2, br, 8), lambda i: (0, 0, i, 0)),
      ],
      out_specs=[
          pl.BlockSpec((br, HD), lambda i: (i, 0)),
          pl.BlockSpec((br, HD), lambda i: (i, 0)),
          pl.BlockSpec((br, 1), lambda i: (i, 0)),
          pl.BlockSpec((br, 1), lambda i: (i, 0)),
      ],
      out_shape=[
          jax.ShapeDtypeStruct((N, HD), jnp.float32),
          jax.ShapeDtypeStruct((N, HD), jnp.float32),
          jax.ShapeDtypeStruct((N, 1), jnp.float32),
          jax.ShapeDtypeStruct((N, 1), jnp.float32),
      ],
  )(h, WA, WB, degs)


def _tc_mid(PA, PB, ri, ro, bA, bB, gA, gB, beA, beB, Wq, br, two_out):
  N, HD = ri.shape[0], PA.shape[2]
  Dtot = 2.0 * HD

  def body(pa_ref, pb_ref, ri_ref, ro_ref, ba_ref, bb_ref, ga_ref, gb_ref,
           bea_ref, beb_ref, *rest):
    w_refs = rest[:len(Wq)]
    y_refs = rest[len(Wq):]
    ri_v = ri_ref[...]
    xA = (pa_ref[0] + pa_ref[1]) * ri_v + ba_ref[...]
    xB = (pb_ref[0] + pb_ref[1]) * ri_v + bb_ref[...]
    mu = (jnp.sum(xA, -1, keepdims=True) + jnp.sum(xB, -1, keepdims=True)) / Dtot
    dA = xA - mu
    dB = xB - mu
    var = (jnp.sum(dA * dA, -1, keepdims=True)
           + jnp.sum(dB * dB, -1, keepdims=True)) / Dtot
    rs = lax.rsqrt(var + 1e-5)
    xA = jnp.maximum(dA * rs * ga_ref[...] + bea_ref[...], 0.0)
    xB = jnp.maximum(dB * rs * gb_ref[...] + beb_ref[...], 0.0)
    ro_v = ro_ref[...]
    y_refs[0][...] = (jnp.dot(xA, w_refs[0][...], preferred_element_type=jnp.float32)
                      + jnp.dot(xB, w_refs[1][...], preferred_element_type=jnp.float32)) * ro_v
    if two_out:
      y_refs[1][...] = (jnp.dot(xA, w_refs[2][...], preferred_element_type=jnp.float32)
                        + jnp.dot(xB, w_refs[3][...], preferred_element_type=jnp.float32)) * ro_v

  n_out = 2 if two_out else 1
  Dn = Wq[0].shape[1]
  return pl.pallas_call(
      body,
      grid=(N // br,),
      in_specs=(
          [pl.BlockSpec((NC, br, HD), lambda i: (0, i, 0))] * 2
          + [pl.BlockSpec((br, 1), lambda i: (i, 0))] * 2
          + [pl.BlockSpec((1, HD), lambda i: (0, 0))] * 6
          + [pl.BlockSpec((HD, Dn), lambda i: (0, 0))] * len(Wq)
      ),
      out_specs=[pl.BlockSpec((br, Dn), lambda i: (i, 0))] * n_out,
      out_shape=[jax.ShapeDtypeStruct((N, Dn), jnp.float32)] * n_out,
  )(PA, PB, ri, ro, bA.reshape(1, HD), bB.reshape(1, HD),
    gA.reshape(1, HD), gB.reshape(1, HD), beA.reshape(1, HD),
    beB.reshape(1, HD), *Wq)


def _tc_last(P, ri, b2, br):
  N, D = ri.shape[0], P.shape[2]

  def body(p_ref, ri_ref, b_ref, y_ref):
    y_ref[...] = (p_ref[0] + p_ref[1]) * ri_ref[...] + b_ref[...]

  return pl.pallas_call(
      body,
      grid=(N // br,),
      in_specs=[
          pl.BlockSpec((NC, br, D), lambda i: (0, i, 0)),
          pl.BlockSpec((br, 1), lambda i: (i, 0)),
          pl.BlockSpec((1, D), lambda i: (0, 0)),
      ],
      out_specs=pl.BlockSpec((br, D), lambda i: (i, 0)),
      out_shape=jax.ShapeDtypeStruct((N, D), jnp.float32),
  )(P, ri, b2.reshape(1, D))


def kernel(h, edge_index, W0, b0, g0, be0, W1, b1, g1, be1, W2, b2):
  N = h.shape[0]
  E = edge_index.shape[1]
  nch_tot = -(-E // (NS * CH)) + 1
  nch_tot += nch_tot % 2
  nch_f = int(nch_tot * FRAC) // 2 * 2
  nch_s = nch_tot - nch_f
  cap_f, cap_s = nch_f * CH, nch_s * CH
  F = NS * cap_f
  src = edge_index[0]
  dst = edge_index[1]

  R = NS * nch_f + (NS - 1) * nch_s + nch_f
  pad = R * CH - E
  sd = jnp.concatenate([src, jnp.full((pad,), N, jnp.int32)]).reshape(R, CH)
  dd = jnp.concatenate([dst, jnp.full((pad,), N, jnp.int32)]).reshape(R, CH)
  sg = sd

  rows_acc = _rows_acc(N)
  ones8 = jnp.ones((CH, 8), jnp.float32)
  zer8 = jnp.zeros((rows_acc, 8), jnp.float32)

  degs = _make_deg(N, nch_f, nch_s, CF)(sd, dd, ones8, zer8)

  HD = W0.shape[1] // 2
  agg = _make_agg(N, nch_f, nch_s, CF, HD)
  br = 1000 if N % 1000 == 0 else N // NS

  y0A, y0B, ro, ri = _tc_first(h, W0[:, :HD], W0[:, HD:], degs, br)
  P0A = agg(y0A, sg, dd)
  P0B = agg(y0B, sg, dd)
  Wq1 = (W1[:HD, :HD], W1[HD:, :HD], W1[:HD, HD:], W1[HD:, HD:])
  y1A, y1B = _tc_mid(P0A, P0B, ri, ro, b0[:HD], b0[HD:], g0[:HD], g0[HD:],
                     be0[:HD], be0[HD:], Wq1, br, two_out=True)
  P1A = agg(y1A, sg, dd)
  P1B = agg(y1B, sg, dd)
  Wq2 = (W2[:HD], W2[HD:])
  (y2,) = _tc_mid(P1A, P1B, ri, ro, b1[:HD], b1[HD:], g1[:HD], g1[HD:],
                  be1[:HD], be1[HD:], Wq2, br, two_out=False)
  P2 = agg(y2, sg, dd)
  return _tc_last(P2, ri, b2, br)


# speedup vs baseline: 1.0333x; 1.0039x over previous
import functools

import jax
import jax.numpy as jnp
from jax import lax
from jax.experimental import pallas as pl
from jax.experimental.pallas import tpu as pltpu
from jax.experimental.pallas import tpu_sc as plsc

NC = 2
NS = 16
CH = 128
NW = NC * NS
CF = 0
FRAC = 0.79


def _rows_acc(N):
  return ((N + 1 + NS * 8 - 1) // (NS * 8)) * 8


def _make_deg(N, nch, nch_s, cf):
  rows_acc = _rows_acc(N)
  nacc = rows_acc * NS
  mesh = plsc.VectorSubcoreMesh(core_axis_name="c", subcore_axis_name="s",
                                num_cores=NC, num_subcores=NS)

  @functools.partial(
      pl.kernel, mesh=mesh,
      compiler_params=pltpu.CompilerParams(use_tc_tiling_on_sc=False),
      out_type=jax.ShapeDtypeStruct((NC, 2, nacc, 8), jnp.float32),
      scratch_types=[
          pltpu.VMEM((nch, CH), jnp.int32),
          pltpu.VMEM((nch, CH), jnp.int32),
          pltpu.VMEM((CH, 8), jnp.float32),
          pltpu.VMEM((rows_acc, 8), jnp.float32),
          pltpu.VMEM_SHARED((nacc, 8), jnp.float32),
          pltpu.VMEM_SHARED((nacc, 8), jnp.float32),
      ],
  )
  def deg(sd_hbm, dd_hbm, ones_hbm, zer_hbm, out_hbm,
          sidx, didx, obuf, zbuf, acc_s, acc_d):
    c = lax.axis_index("c")
    s = lax.axis_index("s")
    my_nch = jnp.where(c == cf, nch, nch_s)
    row_off = jnp.where(c == cf, s * nch, NS * nch + s * nch_s)
    pltpu.sync_copy(ones_hbm, obuf)
    pltpu.sync_copy(zer_hbm, zbuf)
    base = s * rows_acc
    pltpu.sync_copy(zbuf, acc_s.at[pl.ds(base, rows_acc)])
    pltpu.sync_copy(zbuf, acc_d.at[pl.ds(base, rows_acc)])
    plsc.subcore_barrier()
    pltpu.sync_copy(sd_hbm.at[pl.ds(row_off, nch)], sidx)
    pltpu.sync_copy(dd_hbm.at[pl.ds(row_off, nch)], didx)

    def chunk(j, _):
      pltpu.sync_copy(obuf, acc_s.at[sidx.at[j]], add=True)
      pltpu.sync_copy(obuf, acc_d.at[didx.at[j]], add=True)
      return 0

    lax.fori_loop(0, my_nch, chunk, 0)
    plsc.subcore_barrier()
    pltpu.sync_copy(acc_s.at[pl.ds(base, rows_acc)], zbuf)
    pltpu.sync_copy(zbuf, out_hbm.at[c, 0, pl.ds(base, rows_acc)])
    pltpu.sync_copy(acc_d.at[pl.ds(base, rows_acc)], zbuf)
    pltpu.sync_copy(zbuf, out_hbm.at[c, 1, pl.ds(base, rows_acc)])

  return deg


def _make_agg(N, nch, nch_s, cf, D):
  rows_acc = _rows_acc(N)
  nacc = rows_acc * NS
  mesh = plsc.VectorSubcoreMesh(core_axis_name="c", subcore_axis_name="s",
                                num_cores=NC, num_subcores=NS)

  @functools.partial(
      pl.kernel, mesh=mesh,
      compiler_params=pltpu.CompilerParams(use_tc_tiling_on_sc=False),
      out_type=jax.ShapeDtypeStruct((NC, nacc, D), jnp.float32),
      scratch_types=[
          pltpu.VMEM((nch, CH), jnp.int32),
          pltpu.VMEM((nch, CH), jnp.int32),
          pltpu.VMEM((CH, D), jnp.float32),
          pltpu.VMEM((CH, D), jnp.float32),
          pltpu.VMEM_SHARED((nacc, D), jnp.float32),
          pltpu.SemaphoreType.DMA,
          pltpu.SemaphoreType.DMA,
      ],
  )
  def agg(y_hbm, sg_hbm, dd_hbm, out_hbm,
          sidx, didx, bufa, bufb, acc, sema, semb):
    c = lax.axis_index("c")
    s = lax.axis_index("s")
    my_nch = jnp.where(c == cf, nch, nch_s)
    row_off = jnp.where(c == cf, s * nch, NS * nch + s * nch_s)
    z16 = jnp.zeros((16,), jnp.float32)

    def zrow(r, _):
      for kk in range(D // 16):
        bufa[r, pl.ds(kk * 16, 16)] = z16
      return 0

    lax.fori_loop(0, CH, zrow, 0)
    base = s * rows_acc
    nfull, rem = divmod(rows_acc, CH)
    for k in range(nfull):
      pltpu.sync_copy(bufa, acc.at[pl.ds(base + k * CH, CH)])
    if rem:
      pltpu.sync_copy(bufa.at[pl.ds(0, rem)],
                      acc.at[pl.ds(base + nfull * CH, rem)])
    plsc.subcore_barrier()
    pltpu.sync_copy(sg_hbm.at[pl.ds(row_off, nch)], sidx)
    pltpu.sync_copy(dd_hbm.at[pl.ds(row_off, nch)], didx)
    nclamp = jnp.full((16,), N - 1, jnp.int32)

    def crow(r, _):
      for kk in range(CH // 16):
        sidx[r, pl.ds(kk * 16, 16)] = jnp.minimum(
            sidx[r, pl.ds(kk * 16, 16)], nclamp)
      return 0

    lax.fori_loop(0, nch, crow, 0)
    pltpu.async_copy(y_hbm.at[sidx.at[0]], bufa, sema)
    pltpu.async_copy(y_hbm.at[sidx.at[1]], bufb, semb)

    def pair(p, _):
      j = 2 * p
      pltpu.make_async_copy(y_hbm.at[sidx.at[j]], bufa, sema).wait()
      pltpu.sync_copy(bufa, acc.at[didx.at[j]], add=True)
      pltpu.async_copy(y_hbm.at[sidx.at[j + 2]], bufa, sema)
      pltpu.make_async_copy(y_hbm.at[sidx.at[j + 1]], bufb, semb).wait()
      pltpu.sync_copy(bufb, acc.at[didx.at[j + 1]], add=True)
      pltpu.async_copy(y_hbm.at[sidx.at[j + 3]], bufb, semb)
      return 0

    lax.fori_loop(0, my_nch // 2 - 1, pair, 0)
    j = my_nch - 2
    pltpu.make_async_copy(y_hbm.at[sidx.at[j]], bufa, sema).wait()
    pltpu.sync_copy(bufa, acc.at[didx.at[j]], add=True)
    pltpu.make_async_copy(y_hbm.at[sidx.at[j + 1]], bufb, semb).wait()
    pltpu.sync_copy(bufb, acc.at[didx.at[j + 1]], add=True)
    plsc.subcore_barrier()
    for k in range(nfull):
      pltpu.sync_copy(acc.at[pl.ds(base + k * CH, CH)], bufa)
      pltpu.sync_copy(bufa, out_hbm.at[c, pl.ds(base + k * CH, CH)])
    if rem:
      pltpu.sync_copy(acc.at[pl.ds(base + nfull * CH, rem)],
                      bufa.at[pl.ds(0, rem)])
      pltpu.sync_copy(bufa.at[pl.ds(0, rem)],
                      out_hbm.at[c, pl.ds(base + nfull * CH, rem)])

  return agg


def _tc_first(h, WA, WB, degs, br):
  N, HD = h.shape[0], WA.shape[1]

  def body(h_ref, wa_ref, wb_ref, d_ref, ya_ref, yb_ref, ro_ref, ri_ref):
    d = d_ref[...]
    dout = (d[0, 0] + d[1, 0])[:, 0:1]
    din = (d[0, 1] + d[1, 1])[:, 0:1]
    ro = lax.rsqrt(jnp.maximum(dout, 1.0))
    ri = lax.rsqrt(jnp.maximum(din, 1.0))
    x = h_ref[...]
    ya_ref[...] = jnp.dot(x, wa_ref[...], preferred_element_type=jnp.float32) * ro
    yb_ref[...] = jnp.dot(x, wb_ref[...], preferred_element_type=jnp.float32) * ro
    ro_ref[...] = ro
    ri_ref[...] = ri

  Din = h.shape[1]
  return pl.pallas_call(
      body,
      grid=(N // br,),
      in_specs=[
          pl.BlockSpec((br, Din), lambda i: (i, 0)),
          pl.BlockSpec((Din, HD), lambda i: (0, 0)),
          pl.BlockSpec((Din, HD), lambda i: (0, 0)),
          pl.BlockSpec((NC, 2, br, 8), lambda i: (0, 0, i, 0)),
      ],
      out_specs=[
          pl.BlockSpec((br, HD), lambda i: (i, 0)),
          pl.BlockSpec((br, HD), lambda i: (i, 0)),
          pl.BlockSpec((br, 1), lambda i: (i, 0)),
          pl.BlockSpec((br, 1), lambda i: (i, 0)),
      ],
      out_shape=[
          jax.ShapeDtypeStruct((N, HD), jnp.float32),
          jax.ShapeDtypeStruct((N, HD), jnp.float32),
          jax.ShapeDtypeStruct((N, 1), jnp.float32),
          jax.ShapeDtypeStruct((N, 1), jnp.float32),
      ],
  )(h, WA, WB, degs)


def _tc_mid(PA, PB, ri, ro, bA, bB, gA, gB, beA, beB, Wq, br, two_out):
  N, HD = ri.shape[0], PA.shape[2]
  Dtot = 2.0 * HD

  def body(pa_ref, pb_ref, ri_ref, ro_ref, ba_ref, bb_ref, ga_ref, gb_ref,
           bea_ref, beb_ref, *rest):
    w_refs = rest[:len(Wq)]
    y_refs = rest[len(Wq):]
    ri_v = ri_ref[...]
    xA = (pa_ref[0] + pa_ref[1]) * ri_v + ba_ref[...]
    xB = (pb_ref[0] + pb_ref[1]) * ri_v + bb_ref[...]
    mu = (jnp.sum(xA, -1, keepdims=True) + jnp.sum(xB, -1, keepdims=True)) / Dtot
    dA = xA - mu
    dB = xB - mu
    var = (jnp.sum(dA * dA, -1, keepdims=True)
           + jnp.sum(dB * dB, -1, keepdims=True)) / Dtot
    rs = lax.rsqrt(var + 1e-5)
    xA = jnp.maximum(dA * rs * ga_ref[...] + bea_ref[...], 0.0)
    xB = jnp.maximum(dB * rs * gb_ref[...] + beb_ref[...], 0.0)
    ro_v = ro_ref[...]
    y_refs[0][...] = (jnp.dot(xA, w_refs[0][...], preferred_element_type=jnp.float32)
                      + jnp.dot(xB, w_refs[1][...], preferred_element_type=jnp.float32)) * ro_v
    if two_out:
      y_refs[1][...] = (jnp.dot(xA, w_refs[2][...], preferred_element_type=jnp.float32)
                        + jnp.dot(xB, w_refs[3][...], preferred_element_type=jnp.float32)) * ro_v

  n_out = 2 if two_out else 1
  Dn = Wq[0].shape[1]
  return pl.pallas_call(
      body,
      grid=(N // br,),
      in_specs=(
          [pl.BlockSpec((NC, br, HD), lambda i: (0, i, 0))] * 2
          + [pl.BlockSpec((br, 1), lambda i: (i, 0))] * 2
          + [pl.BlockSpec((1, HD), lambda i: (0, 0))] * 6
          + [pl.BlockSpec((HD, Dn), lambda i: (0, 0))] * len(Wq)
      ),
      out_specs=[pl.BlockSpec((br, Dn), lambda i: (i, 0))] * n_out,
      out_shape=[jax.ShapeDtypeStruct((N, Dn), jnp.float32)] * n_out,
  )(PA, PB, ri, ro, bA.reshape(1, HD), bB.reshape(1, HD),
    gA.reshape(1, HD), gB.reshape(1, HD), beA.reshape(1, HD),
    beB.reshape(1, HD), *Wq)


def _tc_last(P, ri, b2, br):
  N, D = ri.shape[0], P.shape[2]

  def body(p_ref, ri_ref, b_ref, y_ref):
    y_ref[...] = (p_ref[0] + p_ref[1]) * ri_ref[...] + b_ref[...]

  return pl.pallas_call(
      body,
      grid=(N // br,),
      in_specs=[
          pl.BlockSpec((NC, br, D), lambda i: (0, i, 0)),
          pl.BlockSpec((br, 1), lambda i: (i, 0)),
          pl.BlockSpec((1, D), lambda i: (0, 0)),
      ],
      out_specs=pl.BlockSpec((br, D), lambda i: (i, 0)),
      out_shape=jax.ShapeDtypeStruct((N, D), jnp.float32),
  )(P, ri, b2.reshape(1, D))


def kernel(h, edge_index, W0, b0, g0, be0, W1, b1, g1, be1, W2, b2):
  N = h.shape[0]
  E = edge_index.shape[1]
  nch_tot = -(-E // (NS * CH)) + 1
  nch_tot += nch_tot % 2
  nch_f = int(nch_tot * FRAC) // 2 * 2
  nch_s = nch_tot - nch_f
  cap_f, cap_s = nch_f * CH, nch_s * CH
  F = NS * cap_f
  src = edge_index[0]
  dst = edge_index[1]

  R = NS * nch_f + (NS - 1) * nch_s + nch_f
  pad = R * CH - E
  sd = jnp.concatenate([src, jnp.full((pad,), N, jnp.int32)]).reshape(R, CH)
  dd = jnp.concatenate([dst, jnp.full((pad,), N, jnp.int32)]).reshape(R, CH)
  sg = sd

  rows_acc = _rows_acc(N)
  ones8 = jnp.ones((CH, 8), jnp.float32)
  zer8 = jnp.zeros((rows_acc, 8), jnp.float32)

  degs = _make_deg(N, nch_f, nch_s, CF)(sd, dd, ones8, zer8)

  HD = W0.shape[1] // 2
  agg = _make_agg(N, nch_f, nch_s, CF, HD)
  br = 1000 if N % 1000 == 0 else N // NS

  y0A, y0B, ro, ri = _tc_first(h, W0[:, :HD], W0[:, HD:], degs, br)
  P0A = agg(y0A, sg, dd)
  P0B = agg(y0B, sg, dd)
  Wq1 = (W1[:HD, :HD], W1[HD:, :HD], W1[:HD, HD:], W1[HD:, HD:])
  y1A, y1B = _tc_mid(P0A, P0B, ri, ro, b0[:HD], b0[HD:], g0[:HD], g0[HD:],
                     be0[:HD], be0[HD:], Wq1, br, two_out=True)
  P1A = agg(y1A, sg, dd)
  P1B = agg(y1B, sg, dd)
  Wq2 = (W2[:HD], W2[HD:])
  (y2,) = _tc_mid(P1A, P1B, ri, ro, b1[:HD], b1[HD:], g1[:HD], g1[HD:],
                  be1[:HD], be1[HD:], Wq2, br, two_out=False)
  P2 = agg(y2, sg, dd)
  return _tc_last(P2, ri, b2, br)
